# Initial kernel scaffold; baseline (speedup 1.0000x reference)
import jax
import jax.numpy as jnp
from jax.experimental import pallas as pl


def kernel(x, edge_index, batch, W_in1, b_in1, W_in2, b_in2, Wl1, Wr1, att1, bias1, Wl2, Wr2, att2, bias2, W_out1, b_out1, W_out2, b_out2):
    raise NotImplementedError("write your pallas kernel here")


# speedup vs baseline: 11.1677x; 11.1677x over previous
import functools

import jax
import jax.numpy as jnp
from jax import lax
from jax.experimental import pallas as pl
from jax.experimental.pallas import tpu as pltpu
from jax.experimental.pallas import tpu_sc as plsc

N = 10000
E = 160000
E2 = E + N
NIN = 128
NHID = 128
HEADS = 2
NOUT = 128
NGRAPHS = 64

N_PAD = 10112
E_PAD = 170240
CH = 128
ROWW = 144
CHUNK = 56
NSLC = CH // 16
RPT = N_PAD // 16
RB = 1264


def _leaky(v, s):
    return jnp.where(v >= 0, v, s * v)


def _mlp_in_body(x_ref, w1_ref, b1_ref, w2_ref, b2_ref, o_ref):
    h = jnp.maximum(jnp.dot(x_ref[...], w1_ref[...],
                            preferred_element_type=jnp.float32) + b1_ref[...], 0.0)
    o_ref[...] = jnp.dot(h, w2_ref[...],
                         preferred_element_type=jnp.float32) + b2_ref[...]


def _mlp_in(x_pad, W1, b1, W2, b2):
    return pl.pallas_call(
        _mlp_in_body,
        grid=(N_PAD // RB,),
        in_specs=[
            pl.BlockSpec((RB, NIN), lambda i: (i, 0)),
            pl.BlockSpec((NIN, NHID), lambda i: (0, 0)),
            pl.BlockSpec((1, NHID), lambda i: (0, 0)),
            pl.BlockSpec((NHID, NHID), lambda i: (0, 0)),
            pl.BlockSpec((1, NHID), lambda i: (0, 0)),
        ],
        out_specs=pl.BlockSpec((RB, NHID), lambda i: (i, 0)),
        out_shape=jax.ShapeDtypeStruct((N_PAD, NHID), jnp.float32),
    )(x_pad, W1, b1.reshape(1, NHID), W2, b2.reshape(1, NHID))


def _proj1_body(h_ref, wl_ref, wr_ref, xl_ref, xr_ref):
    h = h_ref[...]
    xl_ref[...] = jnp.dot(h, wl_ref[...],
                          preferred_element_type=jnp.float32)[None]
    xr_ref[...] = jnp.dot(h, wr_ref[...],
                          preferred_element_type=jnp.float32)[None]


def _proj1(h_pad, Wl, Wr):
    xl, xr = pl.pallas_call(
        _proj1_body,
        grid=(HEADS, N_PAD // RB),
        in_specs=[
            pl.BlockSpec((RB, NHID), lambda c, i: (i, 0)),
            pl.BlockSpec((NHID, NHID), lambda c, i: (0, c)),
            pl.BlockSpec((NHID, NHID), lambda c, i: (0, c)),
        ],
        out_specs=[
            pl.BlockSpec((1, RB, NHID), lambda c, i: (c, i, 0)),
            pl.BlockSpec((1, RB, NHID), lambda c, i: (c, i, 0)),
        ],
        out_shape=[
            jax.ShapeDtypeStruct((HEADS, N_PAD, NHID), jnp.float32),
            jax.ShapeDtypeStruct((HEADS, N_PAD, NHID), jnp.float32),
        ],
    )(h_pad, Wl, Wr)
    return xl, xr


def _mid_body(a_ref, b1_ref, wl_ref, wr_ref, xl_ref, xr_ref):
    a = a_ref[...]
    h0 = _leaky(a[0, :, 0:CH] / (a[0, :, CH:CH + 1] + 1e-16) + b1_ref[0], 0.01)
    h1 = _leaky(a[1, :, 0:CH] / (a[1, :, CH:CH + 1] + 1e-16) + b1_ref[1], 0.01)
    wl = wl_ref[...]
    wr = wr_ref[...]
    xl_ref[...] = (jnp.dot(h0, wl[0:NHID], preferred_element_type=jnp.float32)
                   + jnp.dot(h1, wl[NHID:2 * NHID],
                             preferred_element_type=jnp.float32))
    xr_ref[...] = (jnp.dot(h0, wr[0:NHID], preferred_element_type=jnp.float32)
                   + jnp.dot(h1, wr[NHID:2 * NHID],
                             preferred_element_type=jnp.float32))


def _mid(acc1, bias1, Wl2, Wr2):
    return pl.pallas_call(
        _mid_body,
        grid=(N_PAD // RB,),
        in_specs=[
            pl.BlockSpec((2, RB, ROWW), lambda i: (0, i, 0)),
            pl.BlockSpec((2, 1, NHID), lambda i: (0, 0, 0)),
            pl.BlockSpec((2 * NHID, NHID), lambda i: (0, 0)),
            pl.BlockSpec((2 * NHID, NHID), lambda i: (0, 0)),
        ],
        out_specs=[
            pl.BlockSpec((RB, NHID), lambda i: (i, 0)),
            pl.BlockSpec((RB, NHID), lambda i: (i, 0)),
        ],
        out_shape=[
            jax.ShapeDtypeStruct((N_PAD, NHID), jnp.float32),
            jax.ShapeDtypeStruct((N_PAD, NHID), jnp.float32),
        ],
    )(acc1, bias1.reshape(2, 1, NHID), Wl2, Wr2)


def _final_body(a_ref, b2_ref, batch_ref, w1_ref, b1_ref, w2_ref, b2o_ref,
                o_ref):
    a = a_ref[0] + a_ref[1]
    h2 = _leaky(a[:N, 0:CH] / (a[:N, CH:CH + 1] + 1e-16) + b2_ref[...], 0.01)
    b = batch_ref[...]
    oh = (b == lax.broadcasted_iota(jnp.int32, (N, NGRAPHS), 1))
    oh = oh.astype(jnp.float32)
    sums = lax.dot_general(oh, h2, (((0,), (0,)), ((), ())),
                           preferred_element_type=jnp.float32)
    cnt = jnp.sum(oh, axis=0)
    pooled = sums / jnp.maximum(cnt, 1.0)[:, None]
    t = jnp.maximum(jnp.dot(pooled, w1_ref[...],
                            preferred_element_type=jnp.float32) + b1_ref[...],
                    0.0)
    o_ref[...] = jnp.dot(t, w2_ref[...],
                         preferred_element_type=jnp.float32) + b2o_ref[...]


def _final(acc2, bias2, batch, W_out1, b_out1, W_out2, b_out2):
    return pl.pallas_call(
        _final_body,
        out_shape=jax.ShapeDtypeStruct((NGRAPHS, NOUT), jnp.float32),
    )(acc2, bias2.reshape(1, NHID), batch.reshape(N, 1).astype(jnp.int32),
      W_out1, b_out1.reshape(1, NHID), W_out2, b_out2.reshape(1, NOUT))


def _gat_edges_sc(EC, xl_tab, xr_tab, src_g, dst_g, dst_s, att_stack):
    ET = EC // 16
    NCHUNKS = ET // CHUNK
    mesh = plsc.VectorSubcoreMesh(core_axis_name="c", subcore_axis_name="s")

    @functools.partial(
        pl.kernel,
        out_type=jax.ShapeDtypeStruct((2, N_PAD, ROWW), jnp.float32),
        mesh=mesh,
        compiler_params=pltpu.CompilerParams(
            use_tc_tiling_on_sc=False, needs_layout_passes=False),
        scratch_types=[
            pltpu.VMEM((CHUNK,), jnp.int32),
            pltpu.VMEM((CHUNK,), jnp.int32),
            pltpu.VMEM((CHUNK,), jnp.int32),
            pltpu.VMEM((CHUNK, CH), jnp.float32),
            pltpu.VMEM((CHUNK, CH), jnp.float32),
            pltpu.VMEM((CHUNK, ROWW), jnp.float32),
            pltpu.VMEM((16,), jnp.float32),
            pltpu.VMEM((CH,), jnp.float32),
            pltpu.VMEM_SHARED((N_PAD, ROWW), jnp.float32),
            pltpu.SemaphoreType.DMA,
            pltpu.SemaphoreType.DMA,
        ],
    )
    def k(xl_hbm, xr_hbm, sg_hbm, dg_hbm, ds_hbm, att_hbm, out_hbm,
          isrc, idg, ids, xlbuf, xrbuf, scbuf, rbuf, attv, acc, sem1, sem2):
        cid = lax.axis_index("c")
        sid = lax.axis_index("s")
        pltpu.sync_copy(att_hbm.at[cid], attv)

        def zrow(e, carry):
            for s in range(ROWW // 16):
                scbuf[e, pl.ds(s * 16, 16)] = jnp.zeros((16,), jnp.float32)
            return carry
        lax.fori_loop(0, CHUNK, zrow, 0)

        roff = sid * RPT

        def zacc(i, carry):
            pltpu.sync_copy(scbuf.at[pl.ds(0, 8)],
                            acc.at[pl.ds(roff + i * 8, 8)])
            return carry
        lax.fori_loop(0, RPT // 8, zacc, 0)
        plsc.subcore_barrier()

        attp = [attv[pl.ds(s * 16, 16)] for s in range(NSLC)]
        attn = [a * jnp.float32(0.2) for a in attp]
        lane0 = jnp.arange(16, dtype=jnp.int32) == 0
        col_ex = jnp.full((16,), CH, jnp.int32)
        perms = {s: jnp.arange(16, dtype=jnp.int32) ^ s for s in (8, 4, 2, 1)}
        ebase = cid * EC + sid * ET

        def chunk_body(ci, carry):
            base = ebase + ci * CHUNK
            pltpu.sync_copy(sg_hbm.at[pl.ds(base, CHUNK)], isrc)
            pltpu.sync_copy(dg_hbm.at[pl.ds(base, CHUNK)], idg)
            pltpu.sync_copy(ds_hbm.at[pl.ds(base, CHUNK)], ids)
            cp1 = pltpu.async_copy(xl_hbm.at[isrc], xlbuf, sem1)
            cp2 = pltpu.async_copy(xr_hbm.at[idg], xrbuf, sem2)
            cp1.wait()
            cp2.wait()

            def edge_body(e, c2):
                z = xlbuf[e, pl.ds(0, 16)] + xrbuf[e, pl.ds(0, 16)]
                av = (attp[0] * jnp.maximum(z, 0.0)
                      + attn[0] * jnp.minimum(z, 0.0))
                for s in range(1, NSLC):
                    z = (xlbuf[e, pl.ds(s * 16, 16)]
                         + xrbuf[e, pl.ds(s * 16, 16)])
                    av = av + (attp[s] * jnp.maximum(z, 0.0)
                               + attn[s] * jnp.minimum(z, 0.0))
                for step in (8, 4, 2, 1):
                    rbuf[...] = av
                    av = av + plsc.load_gather(rbuf, [perms[step]])
                exv = jnp.exp(av)
                for s in range(NSLC):
                    scbuf[e, pl.ds(s * 16, 16)] = (
                        xlbuf[e, pl.ds(s * 16, 16)] * exv)
                plsc.store_scatter(
                    scbuf, [jnp.full((16,), e, jnp.int32), col_ex], exv,
                    mask=lane0)
                return c2
            lax.fori_loop(0, CHUNK, edge_body, 0)

            pltpu.sync_copy(scbuf, acc.at[ids], add=True)
            return carry
        lax.fori_loop(0, NCHUNKS, chunk_body, 0)

        plsc.subcore_barrier()
        pltpu.sync_copy(acc.at[pl.ds(roff, RPT)],
                        out_hbm.at[cid, pl.ds(roff, RPT)])

    return k(xl_tab, xr_tab, src_g, dst_g, dst_s, att_stack)


def kernel(x, edge_index, batch, W_in1, b_in1, W_in2, b_in2, Wl1, Wr1, att1,
           bias1, Wl2, Wr2, att2, bias2, W_out1, b_out1, W_out2, b_out2):
    i32 = jnp.int32
    loop = jnp.arange(N, dtype=i32)
    src = jnp.concatenate([edge_index[0].astype(i32), loop])
    dst = jnp.concatenate([edge_index[1].astype(i32), loop])
    npad = E_PAD - E2
    src_p = jnp.concatenate([src, jnp.zeros((npad,), i32)])
    dst_p = jnp.concatenate([dst, jnp.full((npad,), N, i32)])

    sg1 = jnp.concatenate([src_p, src_p + N_PAD])
    dg1 = jnp.concatenate([dst_p, dst_p + N_PAD])
    ds1 = jnp.concatenate([dst_p, dst_p])

    x_pad = jnp.pad(x, ((0, N_PAD - N), (0, 0)))
    h = _mlp_in(x_pad, W_in1, b_in1, W_in2, b_in2)
    xl1, xr1 = _proj1(h, Wl1, Wr1)
    xl1 = xl1.reshape(HEADS * N_PAD, NHID)
    xr1 = xr1.reshape(HEADS * N_PAD, NHID)

    acc1 = _gat_edges_sc(E_PAD, xl1, xr1, sg1, dg1, ds1, att1)

    xl2, xr2 = _mid(acc1, bias1, Wl2, Wr2)

    att2s = jnp.concatenate([att2, att2], axis=0)
    acc2 = _gat_edges_sc(E_PAD // 2, xl2, xr2, src_p, dst_p, dst_p, att2s)

    return _final(acc2, bias2, batch, W_out1, b_out1, W_out2, b_out2)


# speedup vs baseline: 14.9244x; 1.3364x over previous
import functools

import jax
import jax.numpy as jnp
from jax import lax
from jax.experimental import pallas as pl
from jax.experimental.pallas import tpu as pltpu
from jax.experimental.pallas import tpu_sc as plsc

N = 10000
E = 160000
E2 = E + N
NIN = 128
NHID = 128
HEADS = 2
NOUT = 128
NGRAPHS = 64

N_PAD = 10112
E_PAD = 172032
CH = 128
ROWW = 144
CHUNK = 48
SB = 8
LOG_SB = 3
NSLC = CH // 16
RPT = N_PAD // 16
RB = 1264


def _leaky(v, s):
    return jnp.where(v >= 0, v, s * v)


def _mlp_in_body(x_ref, w1_ref, b1_ref, w2_ref, b2_ref, o_ref):
    h = jnp.maximum(jnp.dot(x_ref[...], w1_ref[...],
                            preferred_element_type=jnp.float32) + b1_ref[...], 0.0)
    o_ref[...] = jnp.dot(h, w2_ref[...],
                         preferred_element_type=jnp.float32) + b2_ref[...]


def _mlp_in(x_pad, W1, b1, W2, b2):
    return pl.pallas_call(
        _mlp_in_body,
        grid=(N_PAD // RB,),
        in_specs=[
            pl.BlockSpec((RB, NIN), lambda i: (i, 0)),
            pl.BlockSpec((NIN, NHID), lambda i: (0, 0)),
            pl.BlockSpec((1, NHID), lambda i: (0, 0)),
            pl.BlockSpec((NHID, NHID), lambda i: (0, 0)),
            pl.BlockSpec((1, NHID), lambda i: (0, 0)),
        ],
        out_specs=pl.BlockSpec((RB, NHID), lambda i: (i, 0)),
        out_shape=jax.ShapeDtypeStruct((N_PAD, NHID), jnp.float32),
    )(x_pad, W1, b1.reshape(1, NHID), W2, b2.reshape(1, NHID))


def _proj1_body(h_ref, wl_ref, wr_ref, xl_ref, xr_ref):
    h = h_ref[...]
    xl_ref[...] = jnp.dot(h, wl_ref[...],
                          preferred_element_type=jnp.float32)[None]
    xr_ref[...] = jnp.dot(h, wr_ref[...],
                          preferred_element_type=jnp.float32)[None]


def _proj1(h_pad, Wl, Wr):
    xl, xr = pl.pallas_call(
        _proj1_body,
        grid=(HEADS, N_PAD // RB),
        in_specs=[
            pl.BlockSpec((RB, NHID), lambda c, i: (i, 0)),
            pl.BlockSpec((NHID, NHID), lambda c, i: (0, c)),
            pl.BlockSpec((NHID, NHID), lambda c, i: (0, c)),
        ],
        out_specs=[
            pl.BlockSpec((1, RB, NHID), lambda c, i: (c, i, 0)),
            pl.BlockSpec((1, RB, NHID), lambda c, i: (c, i, 0)),
        ],
        out_shape=[
            jax.ShapeDtypeStruct((HEADS, N_PAD, NHID), jnp.float32),
            jax.ShapeDtypeStruct((HEADS, N_PAD, NHID), jnp.float32),
        ],
    )(h_pad, Wl, Wr)
    return xl, xr


def _mid_body(a_ref, b1_ref, wl_ref, wr_ref, xl_ref, xr_ref):
    a = a_ref[...]
    h0 = _leaky(a[0, :, 0:CH] / (a[0, :, CH:CH + 1] + 1e-16) + b1_ref[0], 0.01)
    h1 = _leaky(a[1, :, 0:CH] / (a[1, :, CH:CH + 1] + 1e-16) + b1_ref[1], 0.01)
    wl = wl_ref[...]
    wr = wr_ref[...]
    xl_ref[...] = (jnp.dot(h0, wl[0:NHID], preferred_element_type=jnp.float32)
                   + jnp.dot(h1, wl[NHID:2 * NHID],
                             preferred_element_type=jnp.float32))
    xr_ref[...] = (jnp.dot(h0, wr[0:NHID], preferred_element_type=jnp.float32)
                   + jnp.dot(h1, wr[NHID:2 * NHID],
                             preferred_element_type=jnp.float32))


def _mid(acc1, bias1, Wl2, Wr2):
    return pl.pallas_call(
        _mid_body,
        grid=(N_PAD // RB,),
        in_specs=[
            pl.BlockSpec((2, RB, ROWW), lambda i: (0, i, 0)),
            pl.BlockSpec((2, 1, NHID), lambda i: (0, 0, 0)),
            pl.BlockSpec((2 * NHID, NHID), lambda i: (0, 0)),
            pl.BlockSpec((2 * NHID, NHID), lambda i: (0, 0)),
        ],
        out_specs=[
            pl.BlockSpec((RB, NHID), lambda i: (i, 0)),
            pl.BlockSpec((RB, NHID), lambda i: (i, 0)),
        ],
        out_shape=[
            jax.ShapeDtypeStruct((N_PAD, NHID), jnp.float32),
            jax.ShapeDtypeStruct((N_PAD, NHID), jnp.float32),
        ],
    )(acc1, bias1.reshape(2, 1, NHID), Wl2, Wr2)


def _final_body(a_ref, b2_ref, batch_ref, w1_ref, b1_ref, w2_ref, b2o_ref,
                o_ref):
    a = a_ref[0] + a_ref[1]
    h2 = _leaky(a[:N, 0:CH] / (a[:N, CH:CH + 1] + 1e-16) + b2_ref[...], 0.01)
    b = batch_ref[...]
    oh = (b == lax.broadcasted_iota(jnp.int32, (N, NGRAPHS), 1))
    oh = oh.astype(jnp.float32)
    sums = lax.dot_general(oh, h2, (((0,), (0,)), ((), ())),
                           preferred_element_type=jnp.float32)
    cnt = jnp.sum(oh, axis=0)
    pooled = sums / jnp.maximum(cnt, 1.0)[:, None]
    t = jnp.maximum(jnp.dot(pooled, w1_ref[...],
                            preferred_element_type=jnp.float32) + b1_ref[...],
                    0.0)
    o_ref[...] = jnp.dot(t, w2_ref[...],
                         preferred_element_type=jnp.float32) + b2o_ref[...]


def _final(acc2, bias2, batch, W_out1, b_out1, W_out2, b_out2):
    return pl.pallas_call(
        _final_body,
        out_shape=jax.ShapeDtypeStruct((NGRAPHS, NOUT), jnp.float32),
    )(acc2, bias2.reshape(1, NHID), batch.reshape(N, 1).astype(jnp.int32),
      W_out1, b_out1.reshape(1, NHID), W_out2, b_out2.reshape(1, NOUT))


def _gat_edges_sc(EC, xl_tab, xr_tab, src_g, dst_g, dst_s, att_stack):
    ET = EC // 16
    NCHUNKS = ET // CHUNK
    NROWS_CORE = EC // CHUNK
    mesh = plsc.VectorSubcoreMesh(core_axis_name="c", subcore_axis_name="s")

    @functools.partial(
        pl.kernel,
        out_type=jax.ShapeDtypeStruct((2, N_PAD, ROWW), jnp.float32),
        mesh=mesh,
        compiler_params=pltpu.CompilerParams(
            use_tc_tiling_on_sc=False, needs_layout_passes=False),
        scratch_types=[
            pltpu.VMEM((2, SB, CHUNK), jnp.int32),
            pltpu.VMEM((2, SB, CHUNK), jnp.int32),
            pltpu.VMEM((2, SB, CHUNK), jnp.int32),
            pltpu.VMEM((2, CHUNK, CH), jnp.float32),
            pltpu.VMEM((2, CHUNK, CH), jnp.float32),
            pltpu.VMEM((CHUNK, ROWW), jnp.float32),
            pltpu.VMEM((2, 16), jnp.float32),
            pltpu.VMEM((CH,), jnp.float32),
            pltpu.VMEM_SHARED((N_PAD, ROWW), jnp.float32),
            pltpu.SemaphoreType.DMA,
            pltpu.SemaphoreType.DMA,
        ],
    )
    def k(xl_hbm, xr_hbm, sg_hbm, dg_hbm, ds_hbm, att_hbm, out_hbm,
          isup, dsup, ssup, xlb, xrb, scbuf, rbuf, attv, acc, sem1, sem2):
        cid = lax.axis_index("c")
        sid = lax.axis_index("s")
        pltpu.sync_copy(att_hbm.at[cid], attv)

        def zrow(e, carry):
            for s in range(ROWW // 16):
                scbuf[e, pl.ds(s * 16, 16)] = jnp.zeros((16,), jnp.float32)
            return carry
        lax.fori_loop(0, CHUNK, zrow, 0)

        roff = sid * RPT

        def zacc(i, carry):
            pltpu.sync_copy(scbuf.at[pl.ds(0, 8)],
                            acc.at[pl.ds(roff + i * 8, 8)])
            return carry
        lax.fori_loop(0, RPT // 8, zacc, 0)
        plsc.subcore_barrier()

        attp = [attv[pl.ds(s * 16, 16)] for s in range(NSLC)]
        attn = [a * jnp.float32(0.2) for a in attp]
        lane0 = jnp.arange(16, dtype=jnp.int32) == 0
        col_ex = jnp.full((16,), CH, jnp.int32)
        perms = {s: jnp.arange(16, dtype=jnp.int32) ^ s for s in (8, 4, 2, 1)}
        rbase = cid * NROWS_CORE + sid * NCHUNKS
        sems = [sem1, sem2]

        def refill(nxt):
            @pl.when((nxt & (SB - 1)) == 0)
            def _():
                sup = lax.shift_right_logical(nxt, LOG_SB) & 1
                pltpu.sync_copy(sg_hbm.at[pl.ds(rbase + nxt, SB)],
                                isup.at[sup])
                pltpu.sync_copy(dg_hbm.at[pl.ds(rbase + nxt, SB)],
                                dsup.at[sup])
                pltpu.sync_copy(ds_hbm.at[pl.ds(rbase + nxt, SB)],
                                ssup.at[sup])

        def issue(ci, slot):
            sup = lax.shift_right_logical(ci, LOG_SB) & 1
            row = ci & (SB - 1)
            pltpu.async_copy(xl_hbm.at[isup.at[sup, row]], xlb.at[slot],
                             sems[slot])
            pltpu.async_copy(xr_hbm.at[dsup.at[sup, row]], xrb.at[slot],
                             sems[slot])

        def drain(slot):
            pltpu.make_async_copy(xl_hbm.at[isup.at[0, 0]], xlb.at[slot],
                                  sems[slot]).wait()
            pltpu.make_async_copy(xr_hbm.at[dsup.at[0, 0]], xrb.at[slot],
                                  sems[slot]).wait()

        def compute(ci, slot):
            sup = lax.shift_right_logical(ci, LOG_SB) & 1
            row = ci & (SB - 1)

            def edge_body(e2, c2):
                for u in range(2):
                    e = e2 * 2 + u
                    z = xlb[slot, e, pl.ds(0, 16)] + xrb[slot, e, pl.ds(0, 16)]
                    av = (attp[0] * jnp.maximum(z, 0.0)
                          + attn[0] * jnp.minimum(z, 0.0))
                    for s in range(1, NSLC):
                        z = (xlb[slot, e, pl.ds(s * 16, 16)]
                             + xrb[slot, e, pl.ds(s * 16, 16)])
                        av = av + (attp[s] * jnp.maximum(z, 0.0)
                                   + attn[s] * jnp.minimum(z, 0.0))
                    lrow = jnp.full((16,), u, jnp.int32)
                    for step in (8, 4, 2, 1):
                        rbuf[u] = av
                        av = av + plsc.load_gather(rbuf, [lrow, perms[step]])
                    exv = jnp.exp(av)
                    for s in range(NSLC):
                        scbuf[e, pl.ds(s * 16, 16)] = (
                            xlb[slot, e, pl.ds(s * 16, 16)] * exv)
                    plsc.store_scatter(
                        scbuf, [jnp.full((16,), e, jnp.int32), col_ex], exv,
                        mask=lane0)
                return c2
            lax.fori_loop(0, CHUNK // 2, edge_body, 0)
            pltpu.sync_copy(scbuf, acc.at[ssup.at[sup, row]], add=True)

        refill(0)
        issue(0, 0)

        def pair_body(p, carry):
            cur = 2 * p
            refill(cur + 1)
            issue(cur + 1, 1)
            drain(0)
            compute(cur, 0)

            @pl.when(cur + 2 < NCHUNKS)
            def _():
                refill(cur + 2)
                issue(cur + 2, 0)
            drain(1)
            compute(cur + 1, 1)
            return carry
        lax.fori_loop(0, NCHUNKS // 2, pair_body, 0)

        plsc.subcore_barrier()
        pltpu.sync_copy(acc.at[pl.ds(roff, RPT)],
                        out_hbm.at[cid, pl.ds(roff, RPT)])

    return k(xl_tab, xr_tab, src_g, dst_g, dst_s, att_stack)


def kernel(x, edge_index, batch, W_in1, b_in1, W_in2, b_in2, Wl1, Wr1, att1,
           bias1, Wl2, Wr2, att2, bias2, W_out1, b_out1, W_out2, b_out2):
    i32 = jnp.int32
    loop = jnp.arange(N, dtype=i32)
    src = jnp.concatenate([edge_index[0].astype(i32), loop])
    dst = jnp.concatenate([edge_index[1].astype(i32), loop])
    npad = E_PAD - E2
    src_p = jnp.concatenate([src, jnp.zeros((npad,), i32)])
    dst_p = jnp.concatenate([dst, jnp.full((npad,), N, i32)])

    sg1 = jnp.concatenate([src_p, src_p + N_PAD])
    dg1 = jnp.concatenate([dst_p, dst_p + N_PAD])
    ds1 = jnp.concatenate([dst_p, dst_p])

    x_pad = jnp.pad(x, ((0, N_PAD - N), (0, 0)))
    h = _mlp_in(x_pad, W_in1, b_in1, W_in2, b_in2)
    xl1, xr1 = _proj1(h, Wl1, Wr1)
    xl1 = xl1.reshape(HEADS * N_PAD, NHID)
    xr1 = xr1.reshape(HEADS * N_PAD, NHID)

    acc1 = _gat_edges_sc(E_PAD, xl1, xr1, sg1.reshape(-1, CHUNK),
                         dg1.reshape(-1, CHUNK), ds1.reshape(-1, CHUNK), att1)

    xl2, xr2 = _mid(acc1, bias1, Wl2, Wr2)

    att2s = jnp.concatenate([att2, att2], axis=0)
    src2d = src_p.reshape(-1, CHUNK)
    dst2d = dst_p.reshape(-1, CHUNK)
    acc2 = _gat_edges_sc(E_PAD // 2, xl2, xr2, src2d, dst2d, dst2d, att2s)

    return _final(acc2, bias2, batch, W_out1, b_out1, W_out2, b_out2)


# speedup vs baseline: 18.3096x; 1.2268x over previous
import functools

import jax
import jax.numpy as jnp
from jax import lax
from jax.experimental import pallas as pl
from jax.experimental.pallas import tpu as pltpu
from jax.experimental.pallas import tpu_sc as plsc

N = 10000
E = 160000
E2 = E + N
NIN = 128
NHID = 128
HEADS = 2
NOUT = 128
NGRAPHS = 64

N_PAD = 10112
E_PAD = 172032
CH = 128
ROWW = 144
CHUNK = 48
SB = 8
LOG_SB = 3
NSLC = CH // 16
RPT = N_PAD // 16
RB = 1264


def _leaky(v, s):
    return jnp.where(v >= 0, v, s * v)


def _mlp_in_body(x_ref, w1_ref, b1_ref, w2_ref, b2_ref, o_ref):
    h = jnp.maximum(jnp.dot(x_ref[...], w1_ref[...],
                            preferred_element_type=jnp.float32) + b1_ref[...], 0.0)
    o_ref[...] = jnp.dot(h, w2_ref[...],
                         preferred_element_type=jnp.float32) + b2_ref[...]


def _mlp_in(x_pad, W1, b1, W2, b2):
    return pl.pallas_call(
        _mlp_in_body,
        grid=(N_PAD // RB,),
        in_specs=[
            pl.BlockSpec((RB, NIN), lambda i: (i, 0)),
            pl.BlockSpec((NIN, NHID), lambda i: (0, 0)),
            pl.BlockSpec((1, NHID), lambda i: (0, 0)),
            pl.BlockSpec((NHID, NHID), lambda i: (0, 0)),
            pl.BlockSpec((1, NHID), lambda i: (0, 0)),
        ],
        out_specs=pl.BlockSpec((RB, NHID), lambda i: (i, 0)),
        out_shape=jax.ShapeDtypeStruct((N_PAD, NHID), jnp.float32),
    )(x_pad, W1, b1.reshape(1, NHID), W2, b2.reshape(1, NHID))


def _proj1_body(h_ref, wl_ref, wr_ref, xl_ref, xr_ref):
    h = h_ref[...]
    xl_ref[...] = jnp.dot(h, wl_ref[...],
                          preferred_element_type=jnp.float32)[None]
    xr_ref[...] = jnp.dot(h, wr_ref[...],
                          preferred_element_type=jnp.float32)[None]


def _proj1(h_pad, Wl, Wr):
    xl, xr = pl.pallas_call(
        _proj1_body,
        grid=(HEADS, N_PAD // RB),
        in_specs=[
            pl.BlockSpec((RB, NHID), lambda c, i: (i, 0)),
            pl.BlockSpec((NHID, NHID), lambda c, i: (0, c)),
            pl.BlockSpec((NHID, NHID), lambda c, i: (0, c)),
        ],
        out_specs=[
            pl.BlockSpec((1, RB, NHID), lambda c, i: (c, i, 0)),
            pl.BlockSpec((1, RB, NHID), lambda c, i: (c, i, 0)),
        ],
        out_shape=[
            jax.ShapeDtypeStruct((HEADS, N_PAD, NHID), jnp.float32),
            jax.ShapeDtypeStruct((HEADS, N_PAD, NHID), jnp.float32),
        ],
    )(h_pad, Wl, Wr)
    return xl, xr


def _mid_body(a_ref, b1_ref, wl_ref, wr_ref, xl_ref, xr_ref):
    a = a_ref[...]
    h0 = _leaky(a[0, :, 0:CH] / (a[0, :, CH:CH + 1] + 1e-16) + b1_ref[0], 0.01)
    h1 = _leaky(a[1, :, 0:CH] / (a[1, :, CH:CH + 1] + 1e-16) + b1_ref[1], 0.01)
    wl = wl_ref[...]
    wr = wr_ref[...]
    xl_ref[...] = (jnp.dot(h0, wl[0:NHID], preferred_element_type=jnp.float32)
                   + jnp.dot(h1, wl[NHID:2 * NHID],
                             preferred_element_type=jnp.float32))
    xr_ref[...] = (jnp.dot(h0, wr[0:NHID], preferred_element_type=jnp.float32)
                   + jnp.dot(h1, wr[NHID:2 * NHID],
                             preferred_element_type=jnp.float32))


def _mid(acc1, bias1, Wl2, Wr2):
    return pl.pallas_call(
        _mid_body,
        grid=(N_PAD // RB,),
        in_specs=[
            pl.BlockSpec((2, RB, ROWW), lambda i: (0, i, 0)),
            pl.BlockSpec((2, 1, NHID), lambda i: (0, 0, 0)),
            pl.BlockSpec((2 * NHID, NHID), lambda i: (0, 0)),
            pl.BlockSpec((2 * NHID, NHID), lambda i: (0, 0)),
        ],
        out_specs=[
            pl.BlockSpec((RB, NHID), lambda i: (i, 0)),
            pl.BlockSpec((RB, NHID), lambda i: (i, 0)),
        ],
        out_shape=[
            jax.ShapeDtypeStruct((N_PAD, NHID), jnp.float32),
            jax.ShapeDtypeStruct((N_PAD, NHID), jnp.float32),
        ],
    )(acc1, bias1.reshape(2, 1, NHID), Wl2, Wr2)


def _final_body(a_ref, b2_ref, batch_ref, w1_ref, b1_ref, w2_ref, b2o_ref,
                o_ref):
    a = a_ref[0] + a_ref[1]
    h2 = _leaky(a[:N, 0:CH] / (a[:N, CH:CH + 1] + 1e-16) + b2_ref[...], 0.01)
    b = batch_ref[...]
    oh = (b == lax.broadcasted_iota(jnp.int32, (N, NGRAPHS), 1))
    oh = oh.astype(jnp.float32)
    sums = lax.dot_general(oh, h2, (((0,), (0,)), ((), ())),
                           preferred_element_type=jnp.float32)
    cnt = jnp.sum(oh, axis=0)
    pooled = sums / jnp.maximum(cnt, 1.0)[:, None]
    t = jnp.maximum(jnp.dot(pooled, w1_ref[...],
                            preferred_element_type=jnp.float32) + b1_ref[...],
                    0.0)
    o_ref[...] = jnp.dot(t, w2_ref[...],
                         preferred_element_type=jnp.float32) + b2o_ref[...]


def _final(acc2, bias2, batch, W_out1, b_out1, W_out2, b_out2):
    return pl.pallas_call(
        _final_body,
        out_shape=jax.ShapeDtypeStruct((NGRAPHS, NOUT), jnp.float32),
    )(acc2, bias2.reshape(1, NHID), batch.reshape(N, 1).astype(jnp.int32),
      W_out1, b_out1.reshape(1, NHID), W_out2, b_out2.reshape(1, NOUT))


def _gat_edges_sc(EC, xl_tab, xr_tab, src_g, dst_g, dst_s, att_stack):
    ET = EC // 16
    NCHUNKS = ET // CHUNK
    NROWS_CORE = EC // CHUNK
    mesh = plsc.VectorSubcoreMesh(core_axis_name="c", subcore_axis_name="s")

    @functools.partial(
        pl.kernel,
        out_type=jax.ShapeDtypeStruct((2, N_PAD, ROWW), jnp.float32),
        mesh=mesh,
        compiler_params=pltpu.CompilerParams(
            use_tc_tiling_on_sc=False, needs_layout_passes=False),
        scratch_types=[
            pltpu.VMEM((2, SB, CHUNK), jnp.int32),
            pltpu.VMEM((2, SB, CHUNK), jnp.int32),
            pltpu.VMEM((2, SB, CHUNK), jnp.int32),
            pltpu.VMEM((2, CHUNK, CH), jnp.float32),
            pltpu.VMEM((2, CHUNK, CH), jnp.float32),
            pltpu.VMEM((CHUNK, ROWW), jnp.float32),
            pltpu.VMEM((CHUNK, 16), jnp.float32),
            pltpu.VMEM((CHUNK,), jnp.float32),
            pltpu.VMEM((CH,), jnp.float32),
            pltpu.VMEM_SHARED((N_PAD, ROWW), jnp.float32),
            pltpu.SemaphoreType.DMA,
            pltpu.SemaphoreType.DMA,
        ],
    )
    def k(xl_hbm, xr_hbm, sg_hbm, dg_hbm, ds_hbm, att_hbm, out_hbm,
          isup, dsup, ssup, xlb, xrb, scbuf, pbuf, exbuf, attv, acc,
          sem1, sem2):
        cid = lax.axis_index("c")
        sid = lax.axis_index("s")
        pltpu.sync_copy(att_hbm.at[cid], attv)

        def zrow(e, carry):
            for s in range(ROWW // 16):
                scbuf[e, pl.ds(s * 16, 16)] = jnp.zeros((16,), jnp.float32)
            return carry
        lax.fori_loop(0, CHUNK, zrow, 0)

        roff = sid * RPT

        def zacc(i, carry):
            pltpu.sync_copy(scbuf.at[pl.ds(0, 8)],
                            acc.at[pl.ds(roff + i * 8, 8)])
            return carry
        lax.fori_loop(0, RPT // 8, zacc, 0)
        plsc.subcore_barrier()

        attp = [attv[pl.ds(s * 16, 16)] for s in range(NSLC)]
        attn = [a * jnp.float32(0.2) for a in attp]
        lanes = jnp.arange(16, dtype=jnp.int32)
        col_ex = jnp.full((16,), CH, jnp.int32)
        rbase = cid * NROWS_CORE + sid * NCHUNKS
        sems = [sem1, sem2]

        def refill(nxt):
            @pl.when((nxt & (SB - 1)) == 0)
            def _():
                sup = lax.shift_right_logical(nxt, LOG_SB) & 1
                pltpu.sync_copy(sg_hbm.at[pl.ds(rbase + nxt, SB)],
                                isup.at[sup])
                pltpu.sync_copy(dg_hbm.at[pl.ds(rbase + nxt, SB)],
                                dsup.at[sup])
                pltpu.sync_copy(ds_hbm.at[pl.ds(rbase + nxt, SB)],
                                ssup.at[sup])

        def issue(ci, slot):
            sup = lax.shift_right_logical(ci, LOG_SB) & 1
            row = ci & (SB - 1)
            pltpu.async_copy(xl_hbm.at[isup.at[sup, row]], xlb.at[slot],
                             sems[slot])
            pltpu.async_copy(xr_hbm.at[dsup.at[sup, row]], xrb.at[slot],
                             sems[slot])

        def drain(slot):
            pltpu.make_async_copy(xl_hbm.at[isup.at[0, 0]], xlb.at[slot],
                                  sems[slot]).wait()
            pltpu.make_async_copy(xr_hbm.at[dsup.at[0, 0]], xrb.at[slot],
                                  sems[slot]).wait()

        def compute(ci, slot):
            sup = lax.shift_right_logical(ci, LOG_SB) & 1
            row = ci & (SB - 1)

            def logit_body(e2, c2):
                for u in range(2):
                    e = e2 * 2 + u
                    z = xlb[slot, e, pl.ds(0, 16)] + xrb[slot, e, pl.ds(0, 16)]
                    av = (attp[0] * jnp.maximum(z, 0.0)
                          + attn[0] * jnp.minimum(z, 0.0))
                    for s in range(1, NSLC):
                        z = (xlb[slot, e, pl.ds(s * 16, 16)]
                             + xrb[slot, e, pl.ds(s * 16, 16)])
                        av = av + (attp[s] * jnp.maximum(z, 0.0)
                                   + attn[s] * jnp.minimum(z, 0.0))
                    pbuf[e] = av
                return c2
            lax.fori_loop(0, CHUNK // 2, logit_body, 0)

            def red_body(g, c2):
                evec = g * 16 + lanes
                tot = plsc.load_gather(pbuf, [evec, jnp.zeros((16,),
                                                             jnp.int32)])
                for r in range(1, 16):
                    tot = tot + plsc.load_gather(
                        pbuf, [evec, jnp.full((16,), r, jnp.int32)])
                exg = jnp.exp(tot)
                exbuf[pl.ds(g * 16, 16)] = exg
                plsc.store_scatter(scbuf, [evec, col_ex], exg)
                return c2
            lax.fori_loop(0, CHUNK // 16, red_body, 0)

            def emit_body(e2, c2):
                for u in range(2):
                    e = e2 * 2 + u
                    exv = plsc.load_gather(exbuf,
                                           [jnp.full((16,), e, jnp.int32)])
                    for s in range(NSLC):
                        scbuf[e, pl.ds(s * 16, 16)] = (
                            xlb[slot, e, pl.ds(s * 16, 16)] * exv)
                return c2
            lax.fori_loop(0, CHUNK // 2, emit_body, 0)

            pltpu.sync_copy(scbuf, acc.at[ssup.at[sup, row]], add=True)

        refill(0)
        issue(0, 0)

        def pair_body(p, carry):
            cur = 2 * p
            refill(cur + 1)
            issue(cur + 1, 1)
            drain(0)
            compute(cur, 0)

            @pl.when(cur + 2 < NCHUNKS)
            def _():
                refill(cur + 2)
                issue(cur + 2, 0)
            drain(1)
            compute(cur + 1, 1)
            return carry
        lax.fori_loop(0, NCHUNKS // 2, pair_body, 0)

        plsc.subcore_barrier()
        pltpu.sync_copy(acc.at[pl.ds(roff, RPT)],
                        out_hbm.at[cid, pl.ds(roff, RPT)])

    return k(xl_tab, xr_tab, src_g, dst_g, dst_s, att_stack)


def kernel(x, edge_index, batch, W_in1, b_in1, W_in2, b_in2, Wl1, Wr1, att1,
           bias1, Wl2, Wr2, att2, bias2, W_out1, b_out1, W_out2, b_out2):
    i32 = jnp.int32
    loop = jnp.arange(N, dtype=i32)
    src = jnp.concatenate([edge_index[0].astype(i32), loop])
    dst = jnp.concatenate([edge_index[1].astype(i32), loop])
    npad = E_PAD - E2
    src_p = jnp.concatenate([src, jnp.zeros((npad,), i32)])
    dst_p = jnp.concatenate([dst, jnp.full((npad,), N, i32)])

    sg1 = jnp.concatenate([src_p, src_p + N_PAD])
    dg1 = jnp.concatenate([dst_p, dst_p + N_PAD])
    ds1 = jnp.concatenate([dst_p, dst_p])

    x_pad = jnp.pad(x, ((0, N_PAD - N), (0, 0)))
    h = _mlp_in(x_pad, W_in1, b_in1, W_in2, b_in2)
    xl1, xr1 = _proj1(h, Wl1, Wr1)
    xl1 = xl1.reshape(HEADS * N_PAD, NHID)
    xr1 = xr1.reshape(HEADS * N_PAD, NHID)

    acc1 = _gat_edges_sc(E_PAD, xl1, xr1, sg1.reshape(-1, CHUNK),
                         dg1.reshape(-1, CHUNK), ds1.reshape(-1, CHUNK), att1)

    xl2, xr2 = _mid(acc1, bias1, Wl2, Wr2)

    att2s = jnp.concatenate([att2, att2], axis=0)
    src2d = src_p.reshape(-1, CHUNK)
    dst2d = dst_p.reshape(-1, CHUNK)
    acc2 = _gat_edges_sc(E_PAD // 2, xl2, xr2, src2d, dst2d, dst2d, att2s)

    return _final(acc2, bias2, batch, W_out1, b_out1, W_out2, b_out2)


# speedup vs baseline: 19.7025x; 1.0761x over previous
import functools

import jax
import jax.numpy as jnp
from jax import lax
from jax.experimental import pallas as pl
from jax.experimental.pallas import tpu as pltpu
from jax.experimental.pallas import tpu_sc as plsc

N = 10000
E = 160000
E2 = E + N
NIN = 128
NHID = 128
HEADS = 2
NOUT = 128
NGRAPHS = 64

N_PAD = 10112
E_PAD = 172032
CH = 128
ROWW = 144
CHUNK = 48
SB = 8
LOG_SB = 3
NSLC = CH // 16
RPT = N_PAD // 16
RB = 1264


def _leaky(v, s):
    return jnp.where(v >= 0, v, s * v)


def _mlp_in_body(x_ref, w1_ref, b1_ref, w2_ref, b2_ref, o_ref):
    h = jnp.maximum(jnp.dot(x_ref[...], w1_ref[...],
                            preferred_element_type=jnp.float32) + b1_ref[...], 0.0)
    o_ref[...] = jnp.dot(h, w2_ref[...],
                         preferred_element_type=jnp.float32) + b2_ref[...]


def _mlp_in(x_pad, W1, b1, W2, b2):
    return pl.pallas_call(
        _mlp_in_body,
        grid=(N_PAD // RB,),
        in_specs=[
            pl.BlockSpec((RB, NIN), lambda i: (i, 0)),
            pl.BlockSpec((NIN, NHID), lambda i: (0, 0)),
            pl.BlockSpec((1, NHID), lambda i: (0, 0)),
            pl.BlockSpec((NHID, NHID), lambda i: (0, 0)),
            pl.BlockSpec((1, NHID), lambda i: (0, 0)),
        ],
        out_specs=pl.BlockSpec((RB, NHID), lambda i: (i, 0)),
        out_shape=jax.ShapeDtypeStruct((N_PAD, NHID), jnp.float32),
    )(x_pad, W1, b1.reshape(1, NHID), W2, b2.reshape(1, NHID))


def _proj1_body(h_ref, wl_ref, wr_ref, xl_ref, xr_ref):
    h = h_ref[...]
    xl_ref[...] = jnp.dot(h, wl_ref[...],
                          preferred_element_type=jnp.float32)[None]
    xr_ref[...] = jnp.dot(h, wr_ref[...],
                          preferred_element_type=jnp.float32)[None]


def _proj1(h_pad, Wl, Wr):
    xl, xr = pl.pallas_call(
        _proj1_body,
        grid=(HEADS, N_PAD // RB),
        in_specs=[
            pl.BlockSpec((RB, NHID), lambda c, i: (i, 0)),
            pl.BlockSpec((NHID, NHID), lambda c, i: (0, c)),
            pl.BlockSpec((NHID, NHID), lambda c, i: (0, c)),
        ],
        out_specs=[
            pl.BlockSpec((1, RB, NHID), lambda c, i: (c, i, 0)),
            pl.BlockSpec((1, RB, NHID), lambda c, i: (c, i, 0)),
        ],
        out_shape=[
            jax.ShapeDtypeStruct((HEADS, N_PAD, NHID), jnp.float32),
            jax.ShapeDtypeStruct((HEADS, N_PAD, NHID), jnp.float32),
        ],
    )(h_pad, Wl, Wr)
    return xl, xr


def _mid_body(a_ref, b1_ref, wl_ref, wr_ref, xl_ref, xr_ref):
    a = a_ref[...]
    h0 = _leaky(a[0, :, 0:CH] / (a[0, :, CH:CH + 1] + 1e-16) + b1_ref[0], 0.01)
    h1 = _leaky(a[1, :, 0:CH] / (a[1, :, CH:CH + 1] + 1e-16) + b1_ref[1], 0.01)
    wl = wl_ref[...]
    wr = wr_ref[...]
    xl_ref[...] = (jnp.dot(h0, wl[0:NHID], preferred_element_type=jnp.float32)
                   + jnp.dot(h1, wl[NHID:2 * NHID],
                             preferred_element_type=jnp.float32))
    xr_ref[...] = (jnp.dot(h0, wr[0:NHID], preferred_element_type=jnp.float32)
                   + jnp.dot(h1, wr[NHID:2 * NHID],
                             preferred_element_type=jnp.float32))


def _mid(acc1, bias1, Wl2, Wr2):
    return pl.pallas_call(
        _mid_body,
        grid=(N_PAD // RB,),
        in_specs=[
            pl.BlockSpec((2, RB, ROWW), lambda i: (0, i, 0)),
            pl.BlockSpec((2, 1, NHID), lambda i: (0, 0, 0)),
            pl.BlockSpec((2 * NHID, NHID), lambda i: (0, 0)),
            pl.BlockSpec((2 * NHID, NHID), lambda i: (0, 0)),
        ],
        out_specs=[
            pl.BlockSpec((RB, NHID), lambda i: (i, 0)),
            pl.BlockSpec((RB, NHID), lambda i: (i, 0)),
        ],
        out_shape=[
            jax.ShapeDtypeStruct((N_PAD, NHID), jnp.float32),
            jax.ShapeDtypeStruct((N_PAD, NHID), jnp.float32),
        ],
    )(acc1, bias1.reshape(2, 1, NHID), Wl2, Wr2)


def _final_body(a_ref, b2_ref, batch_ref, w1_ref, b1_ref, w2_ref, b2o_ref,
                o_ref):
    a = a_ref[0] + a_ref[1]
    h2 = _leaky(a[:N, 0:CH] / (a[:N, CH:CH + 1] + 1e-16) + b2_ref[...], 0.01)
    b = batch_ref[...]
    oh = (b == lax.broadcasted_iota(jnp.int32, (N, NGRAPHS), 1))
    oh = oh.astype(jnp.float32)
    sums = lax.dot_general(oh, h2, (((0,), (0,)), ((), ())),
                           preferred_element_type=jnp.float32)
    cnt = jnp.sum(oh, axis=0)
    pooled = sums / jnp.maximum(cnt, 1.0)[:, None]
    t = jnp.maximum(jnp.dot(pooled, w1_ref[...],
                            preferred_element_type=jnp.float32) + b1_ref[...],
                    0.0)
    o_ref[...] = jnp.dot(t, w2_ref[...],
                         preferred_element_type=jnp.float32) + b2o_ref[...]


def _final(acc2, bias2, batch, W_out1, b_out1, W_out2, b_out2):
    return pl.pallas_call(
        _final_body,
        out_shape=jax.ShapeDtypeStruct((NGRAPHS, NOUT), jnp.float32),
    )(acc2, bias2.reshape(1, NHID), batch.reshape(N, 1).astype(jnp.int32),
      W_out1, b_out1.reshape(1, NHID), W_out2, b_out2.reshape(1, NOUT))


def _gat_edges_sc(EC, xl_tab, xr_tab, src_g, dst_g, dst_s, att_stack):
    ET = EC // 16
    NCHUNKS = ET // CHUNK
    NROWS_CORE = EC // CHUNK
    mesh = plsc.VectorSubcoreMesh(core_axis_name="c", subcore_axis_name="s")

    @functools.partial(
        pl.kernel,
        out_type=jax.ShapeDtypeStruct((2, N_PAD, ROWW), jnp.float32),
        mesh=mesh,
        compiler_params=pltpu.CompilerParams(
            use_tc_tiling_on_sc=False, needs_layout_passes=False),
        scratch_types=[
            pltpu.VMEM((2, SB, CHUNK), jnp.int32),
            pltpu.VMEM((2, SB, CHUNK), jnp.int32),
            pltpu.VMEM((2, SB, CHUNK), jnp.int32),
            pltpu.VMEM((2, CHUNK, CH), jnp.float32),
            pltpu.VMEM((2, CHUNK, CH), jnp.float32),
            pltpu.VMEM((CHUNK, ROWW), jnp.float32),
            pltpu.VMEM((CHUNK, 16), jnp.float32),
            pltpu.VMEM((CHUNK,), jnp.float32),
            pltpu.VMEM((CH,), jnp.float32),
            pltpu.VMEM_SHARED((N_PAD, ROWW), jnp.float32),
            pltpu.SemaphoreType.DMA,
            pltpu.SemaphoreType.DMA,
            pltpu.SemaphoreType.DMA,
        ],
    )
    def k(xl_hbm, xr_hbm, sg_hbm, dg_hbm, ds_hbm, att_hbm, out_hbm,
          isup, dsup, ssup, xlb, xrb, scbuf, pbuf, exbuf, attv, acc,
          sem1, sem2, sem3):
        cid = lax.axis_index("c")
        sid = lax.axis_index("s")
        pltpu.sync_copy(att_hbm.at[cid], attv)

        def zrow(e, carry):
            for s in range(ROWW // 16):
                scbuf[e, pl.ds(s * 16, 16)] = jnp.zeros((16,), jnp.float32)
            return carry
        lax.fori_loop(0, CHUNK, zrow, 0)

        roff = sid * RPT

        def zacc(i, carry):
            pltpu.sync_copy(scbuf.at[pl.ds(0, 8)],
                            acc.at[pl.ds(roff + i * 8, 8)])
            return carry
        lax.fori_loop(0, RPT // 8, zacc, 0)
        plsc.subcore_barrier()

        attp = [attv[pl.ds(s * 16, 16)] for s in range(NSLC)]
        attn = [a * jnp.float32(0.2) for a in attp]
        lanes = jnp.arange(16, dtype=jnp.int32)
        col_ex = jnp.full((16,), CH, jnp.int32)
        rbase = cid * NROWS_CORE + sid * NCHUNKS
        sems = [sem1, sem2]

        def refill(nxt):
            @pl.when((nxt & (SB - 1)) == 0)
            def _():
                sup = lax.shift_right_logical(nxt, LOG_SB) & 1
                pltpu.sync_copy(sg_hbm.at[pl.ds(rbase + nxt, SB)],
                                isup.at[sup])
                pltpu.sync_copy(dg_hbm.at[pl.ds(rbase + nxt, SB)],
                                dsup.at[sup])
                pltpu.sync_copy(ds_hbm.at[pl.ds(rbase + nxt, SB)],
                                ssup.at[sup])

        def issue(ci, slot):
            sup = lax.shift_right_logical(ci, LOG_SB) & 1
            row = ci & (SB - 1)
            pltpu.async_copy(xl_hbm.at[isup.at[sup, row]], xlb.at[slot],
                             sems[slot])
            pltpu.async_copy(xr_hbm.at[dsup.at[sup, row]], xrb.at[slot],
                             sems[slot])

        def drain(slot):
            pltpu.make_async_copy(xl_hbm.at[isup.at[0, 0]], xlb.at[slot],
                                  sems[slot]).wait()
            pltpu.make_async_copy(xr_hbm.at[dsup.at[0, 0]], xrb.at[slot],
                                  sems[slot]).wait()

        def compute(ci, slot):
            sup = lax.shift_right_logical(ci, LOG_SB) & 1
            row = ci & (SB - 1)

            def logit_body(e2, c2):
                for u in range(4):
                    e = e2 * 4 + u
                    z = xlb[slot, e, pl.ds(0, 16)] + xrb[slot, e, pl.ds(0, 16)]
                    av = (attp[0] * jnp.maximum(z, 0.0)
                          + attn[0] * jnp.minimum(z, 0.0))
                    for s in range(1, NSLC):
                        z = (xlb[slot, e, pl.ds(s * 16, 16)]
                             + xrb[slot, e, pl.ds(s * 16, 16)])
                        av = av + (attp[s] * jnp.maximum(z, 0.0)
                                   + attn[s] * jnp.minimum(z, 0.0))
                    pbuf[e] = av
                return c2
            lax.fori_loop(0, CHUNK // 4, logit_body, 0)

            @pl.when(ci > 0)
            def _():
                pltpu.make_async_copy(scbuf, acc.at[ssup.at[0, 0]],
                                      sem3).wait()

            def red_body(g, c2):
                evec = g * 16 + lanes
                tot = plsc.load_gather(pbuf, [evec, jnp.zeros((16,),
                                                             jnp.int32)])
                for r in range(1, 16):
                    tot = tot + plsc.load_gather(
                        pbuf, [evec, jnp.full((16,), r, jnp.int32)])
                exg = jnp.exp(tot)
                exbuf[pl.ds(g * 16, 16)] = exg
                plsc.store_scatter(scbuf, [evec, col_ex], exg)
                return c2
            lax.fori_loop(0, CHUNK // 16, red_body, 0)

            def emit_body(e2, c2):
                for u in range(4):
                    e = e2 * 4 + u
                    exv = plsc.load_gather(exbuf,
                                           [jnp.full((16,), e, jnp.int32)])
                    for s in range(NSLC):
                        scbuf[e, pl.ds(s * 16, 16)] = (
                            xlb[slot, e, pl.ds(s * 16, 16)] * exv)
                return c2
            lax.fori_loop(0, CHUNK // 4, emit_body, 0)

            pltpu.async_copy(scbuf, acc.at[ssup.at[sup, row]], sem3,
                             add=True)

        refill(0)
        issue(0, 0)

        def pair_body(p, carry):
            cur = 2 * p
            refill(cur + 1)
            issue(cur + 1, 1)
            drain(0)
            compute(cur, 0)

            @pl.when(cur + 2 < NCHUNKS)
            def _():
                refill(cur + 2)
                issue(cur + 2, 0)
            drain(1)
            compute(cur + 1, 1)
            return carry
        lax.fori_loop(0, NCHUNKS // 2, pair_body, 0)

        pltpu.make_async_copy(scbuf, acc.at[ssup.at[0, 0]], sem3).wait()
        plsc.subcore_barrier()
        pltpu.sync_copy(acc.at[pl.ds(roff, RPT)],
                        out_hbm.at[cid, pl.ds(roff, RPT)])

    return k(xl_tab, xr_tab, src_g, dst_g, dst_s, att_stack)


def kernel(x, edge_index, batch, W_in1, b_in1, W_in2, b_in2, Wl1, Wr1, att1,
           bias1, Wl2, Wr2, att2, bias2, W_out1, b_out1, W_out2, b_out2):
    i32 = jnp.int32
    loop = jnp.arange(N, dtype=i32)
    src = jnp.concatenate([edge_index[0].astype(i32), loop])
    dst = jnp.concatenate([edge_index[1].astype(i32), loop])
    npad = E_PAD - E2
    src_p = jnp.concatenate([src, jnp.zeros((npad,), i32)])
    dst_p = jnp.concatenate([dst, jnp.full((npad,), N, i32)])

    sg1 = jnp.concatenate([src_p, src_p + N_PAD])
    dg1 = jnp.concatenate([dst_p, dst_p + N_PAD])
    ds1 = jnp.concatenate([dst_p, dst_p])

    x_pad = jnp.pad(x, ((0, N_PAD - N), (0, 0)))
    h = _mlp_in(x_pad, W_in1, b_in1, W_in2, b_in2)
    xl1, xr1 = _proj1(h, Wl1, Wr1)
    xl1 = xl1.reshape(HEADS * N_PAD, NHID)
    xr1 = xr1.reshape(HEADS * N_PAD, NHID)

    acc1 = _gat_edges_sc(E_PAD, xl1, xr1, sg1.reshape(-1, CHUNK),
                         dg1.reshape(-1, CHUNK), ds1.reshape(-1, CHUNK), att1)

    xl2, xr2 = _mid(acc1, bias1, Wl2, Wr2)

    att2s = jnp.concatenate([att2, att2], axis=0)
    src2d = src_p.reshape(-1, CHUNK)
    dst2d = dst_p.reshape(-1, CHUNK)
    acc2 = _gat_edges_sc(E_PAD // 2, xl2, xr2, src2d, dst2d, dst2d, att2s)

    return _final(acc2, bias2, batch, W_out1, b_out1, W_out2, b_out2)
